# Initial kernel scaffold; baseline (speedup 1.0000x reference)
import jax
import jax.numpy as jnp
from jax import lax
from jax.experimental import pallas as pl
from jax.experimental.pallas import tpu as pltpu
from jax.experimental.pallas import tpu_sc as plsc

NC = 2
NS = 16
LANES = 16
NW = NC * NS
EMBED = 64
CHUNK = 128


def _mf_body(user_hbm, itemp_hbm, itemn_hbm, users_hbm, items_hbm,
             pscore_hbm, nscore_hbm,
             idxu, idxp, idxn, urows, prows, nrows, psc, nsc, sem):
    wid = lax.axis_index("s") * NC + lax.axis_index("c")
    n_per_w = user_hbm.shape[0] // NW
    nchunk = n_per_w // CHUNK
    base_w = wid * n_per_w

    def chunk_body(g, carry):
        base = pl.multiple_of(base_w + g * CHUNK, CHUNK)
        pltpu.sync_copy(user_hbm.at[pl.ds(base, CHUNK)], idxu)
        pltpu.sync_copy(itemp_hbm.at[pl.ds(base, CHUNK)], idxp)
        pltpu.sync_copy(itemn_hbm.at[pl.ds(base, CHUNK)], idxn)
        cu = pltpu.async_copy(users_hbm.at[idxu], urows, sem)
        cp = pltpu.async_copy(items_hbm.at[idxp], prows, sem)
        cn = pltpu.async_copy(items_hbm.at[idxn], nrows, sem)
        cu.wait()
        cp.wait()
        cn.wait()
        for grp in range(CHUNK // LANES):
            rows = grp * LANES + lax.iota(jnp.int32, LANES)

            def dbody(d, acc):
                ap, an = acc
                cols = jnp.full((LANES,), 0, jnp.int32) + d
                uu = plsc.load_gather(urows, [rows, cols])
                pp = plsc.load_gather(prows, [rows, cols])
                nn = plsc.load_gather(nrows, [rows, cols])
                return ap + uu * pp, an + uu * nn

            ap, an = lax.fori_loop(
                0, EMBED, dbody,
                (jnp.zeros((LANES,), jnp.float32),
                 jnp.zeros((LANES,), jnp.float32)),
                unroll=16)
            psc[pl.ds(grp * LANES, LANES)] = ap
            nsc[pl.ds(grp * LANES, LANES)] = an
        pltpu.sync_copy(psc, pscore_hbm.at[pl.ds(base, CHUNK)])
        pltpu.sync_copy(nsc, nscore_hbm.at[pl.ds(base, CHUNK)])
        return carry

    lax.fori_loop(0, nchunk, chunk_body, 0)


def kernel(user, item_p, item_n, users, items):
    B, L = user.shape
    N = B * L
    uf = user.reshape(N)
    pf = item_p.reshape(N)
    nf = item_n.reshape(N)
    mesh = plsc.VectorSubcoreMesh(core_axis_name="c", subcore_axis_name="s")
    f = pl.kernel(
        _mf_body,
        out_type=(jax.ShapeDtypeStruct((N,), jnp.float32),
                  jax.ShapeDtypeStruct((N,), jnp.float32)),
        mesh=mesh,
        scratch_types=[
            pltpu.VMEM((CHUNK,), jnp.int32),
            pltpu.VMEM((CHUNK,), jnp.int32),
            pltpu.VMEM((CHUNK,), jnp.int32),
            pltpu.VMEM((CHUNK, EMBED), jnp.float32),
            pltpu.VMEM((CHUNK, EMBED), jnp.float32),
            pltpu.VMEM((CHUNK, EMBED), jnp.float32),
            pltpu.VMEM((CHUNK,), jnp.float32),
            pltpu.VMEM((CHUNK,), jnp.float32),
            pltpu.SemaphoreType.DMA,
        ],
    )
    p_score, n_score = f(uf, pf, nf, users, items)
    return p_score.reshape(B, L), n_score.reshape(B, L)


# speedup vs baseline: 2.5327x; 2.5327x over previous
import jax
import jax.numpy as jnp
from jax import lax
from jax.experimental import pallas as pl
from jax.experimental.pallas import tpu as pltpu
from jax.experimental.pallas import tpu_sc as plsc

NC = 2
NS = 16
LANES = 16
NW = NC * NS
EMBED = 64
CHUNK = 128


def _mf_body(user_hbm, itemp_hbm, itemn_hbm, users_hbm, items_hbm,
             pscore_hbm, nscore_hbm,
             idxu, idxp, idxn, urows, prows, nrows, psc, nsc, sem):
    wid = lax.axis_index("s") * NC + lax.axis_index("c")
    n_per_w = user_hbm.shape[0] // NW
    nchunk = n_per_w // CHUNK
    base_w = wid * n_per_w

    def chunk_body(g, carry):
        base = pl.multiple_of(base_w + g * CHUNK, CHUNK)
        pltpu.sync_copy(user_hbm.at[pl.ds(base, CHUNK)], idxu)
        pltpu.sync_copy(itemp_hbm.at[pl.ds(base, CHUNK)], idxp)
        pltpu.sync_copy(itemn_hbm.at[pl.ds(base, CHUNK)], idxn)
        cu = pltpu.async_copy(users_hbm.at[idxu], urows, sem)
        cp = pltpu.async_copy(items_hbm.at[idxp], prows, sem)
        cn = pltpu.async_copy(items_hbm.at[idxn], nrows, sem)
        cu.wait()
        cp.wait()
        cn.wait()
        last_lane = lax.iota(jnp.int32, LANES) == (LANES - 1)

        def row_body(r, carry):
            accp = None
            accn = None
            for k in range(EMBED // LANES):
                uu = urows[r, pl.ds(k * LANES, LANES)]
                pp = prows[r, pl.ds(k * LANES, LANES)]
                nn = nrows[r, pl.ds(k * LANES, LANES)]
                accp = uu * pp if accp is None else accp + uu * pp
                accn = uu * nn if accn is None else accn + uu * nn
            ridx = jnp.full((LANES,), 0, jnp.int32) + r
            plsc.store_scatter(psc, [ridx], plsc.cumsum(accp), mask=last_lane)
            plsc.store_scatter(nsc, [ridx], plsc.cumsum(accn), mask=last_lane)
            return carry

        lax.fori_loop(0, CHUNK, row_body, 0, unroll=4)
        pltpu.sync_copy(psc, pscore_hbm.at[pl.ds(base, CHUNK)])
        pltpu.sync_copy(nsc, nscore_hbm.at[pl.ds(base, CHUNK)])
        return carry

    lax.fori_loop(0, nchunk, chunk_body, 0)


def kernel(user, item_p, item_n, users, items):
    B, L = user.shape
    N = B * L
    uf = user.reshape(N)
    pf = item_p.reshape(N)
    nf = item_n.reshape(N)
    mesh = plsc.VectorSubcoreMesh(core_axis_name="c", subcore_axis_name="s")
    f = pl.kernel(
        _mf_body,
        out_type=(jax.ShapeDtypeStruct((N,), jnp.float32),
                  jax.ShapeDtypeStruct((N,), jnp.float32)),
        mesh=mesh,
        compiler_params=pltpu.CompilerParams(needs_layout_passes=False,
                                             use_tc_tiling_on_sc=False),
        scratch_types=[
            pltpu.VMEM((CHUNK,), jnp.int32),
            pltpu.VMEM((CHUNK,), jnp.int32),
            pltpu.VMEM((CHUNK,), jnp.int32),
            pltpu.VMEM((CHUNK, EMBED), jnp.float32),
            pltpu.VMEM((CHUNK, EMBED), jnp.float32),
            pltpu.VMEM((CHUNK, EMBED), jnp.float32),
            pltpu.VMEM((CHUNK,), jnp.float32),
            pltpu.VMEM((CHUNK,), jnp.float32),
            pltpu.SemaphoreType.DMA,
        ],
    )
    p_score, n_score = f(uf, pf, nf, users, items)
    return p_score.reshape(B, L), n_score.reshape(B, L)


# speedup vs baseline: 2.7916x; 1.1022x over previous
import jax
import jax.numpy as jnp
from jax import lax
from jax.experimental import pallas as pl
from jax.experimental.pallas import tpu as pltpu
from jax.experimental.pallas import tpu_sc as plsc

NC = 2
NS = 16
LANES = 16
NW = NC * NS
EMBED = 64
CHUNK = 256


def _mf_body(user_hbm, itemp_hbm, itemn_hbm, users_hbm, items_hbm,
             pscore_hbm, nscore_hbm,
             idxu, idxp, idxn, urows, prows, nrows, psc, nsc, sem):
    wid = lax.axis_index("s") * NC + lax.axis_index("c")
    n_per_w = user_hbm.shape[0] // NW
    nchunk = n_per_w // CHUNK
    base_w = wid * n_per_w

    def chunk_body(g, carry):
        base = pl.multiple_of(base_w + g * CHUNK, CHUNK)
        pltpu.sync_copy(user_hbm.at[pl.ds(base, CHUNK)], idxu)
        pltpu.sync_copy(itemp_hbm.at[pl.ds(base, CHUNK)], idxp)
        pltpu.sync_copy(itemn_hbm.at[pl.ds(base, CHUNK)], idxn)
        cu = pltpu.async_copy(users_hbm.at[idxu], urows, sem)
        cp = pltpu.async_copy(items_hbm.at[idxp], prows, sem)
        cn = pltpu.async_copy(items_hbm.at[idxn], nrows, sem)
        cu.wait()
        cp.wait()
        cn.wait()
        last_lane = lax.iota(jnp.int32, LANES) == (LANES - 1)

        def row_body(r, carry):
            accp = None
            accn = None
            for k in range(EMBED // LANES):
                uu = urows[r, pl.ds(k * LANES, LANES)]
                pp = prows[r, pl.ds(k * LANES, LANES)]
                nn = nrows[r, pl.ds(k * LANES, LANES)]
                accp = uu * pp if accp is None else accp + uu * pp
                accn = uu * nn if accn is None else accn + uu * nn
            ridx = jnp.full((LANES,), 0, jnp.int32) + r
            plsc.store_scatter(psc, [ridx], plsc.cumsum(accp), mask=last_lane)
            plsc.store_scatter(nsc, [ridx], plsc.cumsum(accn), mask=last_lane)
            return carry

        lax.fori_loop(0, CHUNK, row_body, 0, unroll=4)
        pltpu.sync_copy(psc, pscore_hbm.at[pl.ds(base, CHUNK)])
        pltpu.sync_copy(nsc, nscore_hbm.at[pl.ds(base, CHUNK)])
        return carry

    lax.fori_loop(0, nchunk, chunk_body, 0)


def kernel(user, item_p, item_n, users, items):
    B, L = user.shape
    N = B * L
    uf = user.reshape(N)
    pf = item_p.reshape(N)
    nf = item_n.reshape(N)
    mesh = plsc.VectorSubcoreMesh(core_axis_name="c", subcore_axis_name="s")
    f = pl.kernel(
        _mf_body,
        out_type=(jax.ShapeDtypeStruct((N,), jnp.float32),
                  jax.ShapeDtypeStruct((N,), jnp.float32)),
        mesh=mesh,
        compiler_params=pltpu.CompilerParams(needs_layout_passes=False,
                                             use_tc_tiling_on_sc=False),
        scratch_types=[
            pltpu.VMEM((CHUNK,), jnp.int32),
            pltpu.VMEM((CHUNK,), jnp.int32),
            pltpu.VMEM((CHUNK,), jnp.int32),
            pltpu.VMEM((CHUNK, EMBED), jnp.float32),
            pltpu.VMEM((CHUNK, EMBED), jnp.float32),
            pltpu.VMEM((CHUNK, EMBED), jnp.float32),
            pltpu.VMEM((CHUNK,), jnp.float32),
            pltpu.VMEM((CHUNK,), jnp.float32),
            pltpu.SemaphoreType.DMA,
        ],
    )
    p_score, n_score = f(uf, pf, nf, users, items)
    return p_score.reshape(B, L), n_score.reshape(B, L)


# speedup vs baseline: 3.4869x; 1.2491x over previous
import jax
import jax.numpy as jnp
from jax import lax
from jax.experimental import pallas as pl
from jax.experimental.pallas import tpu as pltpu
from jax.experimental.pallas import tpu_sc as plsc

NC = 2
NS = 16
LANES = 16
NW = NC * NS
EMBED = 64
CHUNK = 256
NBUF = 2


def _mf_body(user_hbm, itemp_hbm, itemn_hbm, users_hbm, items_hbm,
             pscore_hbm, nscore_hbm,
             idxu, idxp, idxn, urows, prows, nrows, psc, nsc,
             gsem0, gsem1, isem0, isem1, osem0, osem1):
    gsem = (gsem0, gsem1)
    isem = (isem0, isem1)
    osem = (osem0, osem1)
    wid = lax.axis_index("s") * NC + lax.axis_index("c")
    n_per_w = user_hbm.shape[0] // NW
    nchunk = n_per_w // CHUNK
    base_w = wid * n_per_w

    def chunk_base(g):
        return pl.multiple_of(base_w + g * CHUNK, CHUNK)

    def fire_idx(g, b, sync):
        base = chunk_base(g)
        if sync:
            pltpu.sync_copy(user_hbm.at[pl.ds(base, CHUNK)], idxu.at[b])
            pltpu.sync_copy(itemp_hbm.at[pl.ds(base, CHUNK)], idxp.at[b])
            pltpu.sync_copy(itemn_hbm.at[pl.ds(base, CHUNK)], idxn.at[b])
        else:
            pltpu.async_copy(user_hbm.at[pl.ds(base, CHUNK)], idxu.at[b], isem[b])
            pltpu.async_copy(itemp_hbm.at[pl.ds(base, CHUNK)], idxp.at[b], isem[b])
            pltpu.async_copy(itemn_hbm.at[pl.ds(base, CHUNK)], idxn.at[b], isem[b])

    def wait_idx(b):
        pltpu.make_async_copy(user_hbm.at[pl.ds(0, CHUNK)], idxu.at[b], isem[b]).wait()
        pltpu.make_async_copy(itemp_hbm.at[pl.ds(0, CHUNK)], idxp.at[b], isem[b]).wait()
        pltpu.make_async_copy(itemn_hbm.at[pl.ds(0, CHUNK)], idxn.at[b], isem[b]).wait()

    def fire_gather(b):
        pltpu.async_copy(users_hbm.at[idxu.at[b]], urows.at[b], gsem[b])
        pltpu.async_copy(items_hbm.at[idxp.at[b]], prows.at[b], gsem[b])
        pltpu.async_copy(items_hbm.at[idxn.at[b]], nrows.at[b], gsem[b])

    def wait_gather(b):
        pltpu.make_async_copy(users_hbm.at[idxu.at[b]], urows.at[b], gsem[b]).wait()
        pltpu.make_async_copy(items_hbm.at[idxp.at[b]], prows.at[b], gsem[b]).wait()
        pltpu.make_async_copy(items_hbm.at[idxn.at[b]], nrows.at[b], gsem[b]).wait()

    def fire_out(g, b):
        base = chunk_base(g)
        pltpu.async_copy(psc.at[b], pscore_hbm.at[pl.ds(base, CHUNK)], osem[b])
        pltpu.async_copy(nsc.at[b], nscore_hbm.at[pl.ds(base, CHUNK)], osem[b])

    def wait_out(b):
        pltpu.make_async_copy(psc.at[b], pscore_hbm.at[pl.ds(0, CHUNK)], osem[b]).wait()
        pltpu.make_async_copy(nsc.at[b], nscore_hbm.at[pl.ds(0, CHUNK)], osem[b]).wait()

    def compute(b):
        last_lane = lax.iota(jnp.int32, LANES) == (LANES - 1)

        def row_body(r, carry):
            accp = None
            accn = None
            for k in range(EMBED // LANES):
                uu = urows[b, r, pl.ds(k * LANES, LANES)]
                pp = prows[b, r, pl.ds(k * LANES, LANES)]
                nn = nrows[b, r, pl.ds(k * LANES, LANES)]
                accp = uu * pp if accp is None else accp + uu * pp
                accn = uu * nn if accn is None else accn + uu * nn
            ridx = jnp.full((LANES,), 0, jnp.int32) + r
            plsc.store_scatter(psc.at[b], [ridx], plsc.cumsum(accp), mask=last_lane)
            plsc.store_scatter(nsc.at[b], [ridx], plsc.cumsum(accn), mask=last_lane)
            return carry

        lax.fori_loop(0, CHUNK, row_body, 0, unroll=4)

    for b in range(NBUF):
        fire_idx(b, b, sync=True)
        fire_gather(b)

    def outer(i, carry):
        for b in range(NBUF):
            g = i * NBUF + b
            wait_gather(b)

            @pl.when(g >= NBUF)
            def _():
                wait_out(b)

            @pl.when(g < nchunk - NBUF)
            def _():
                fire_idx(g + NBUF, b, sync=False)

            compute(b)
            fire_out(g, b)

            @pl.when(g < nchunk - NBUF)
            def _():
                wait_idx(b)
                fire_gather(b)

        return carry

    lax.fori_loop(0, nchunk // NBUF, outer, 0)

    for b in range(NBUF):
        wait_out(b)


def kernel(user, item_p, item_n, users, items):
    B, L = user.shape
    N = B * L
    uf = user.reshape(N)
    pf = item_p.reshape(N)
    nf = item_n.reshape(N)
    mesh = plsc.VectorSubcoreMesh(core_axis_name="c", subcore_axis_name="s")
    f = pl.kernel(
        _mf_body,
        out_type=(jax.ShapeDtypeStruct((N,), jnp.float32),
                  jax.ShapeDtypeStruct((N,), jnp.float32)),
        mesh=mesh,
        compiler_params=pltpu.CompilerParams(needs_layout_passes=False,
                                             use_tc_tiling_on_sc=False),
        scratch_types=[
            pltpu.VMEM((NBUF, CHUNK), jnp.int32),
            pltpu.VMEM((NBUF, CHUNK), jnp.int32),
            pltpu.VMEM((NBUF, CHUNK), jnp.int32),
            pltpu.VMEM((NBUF, CHUNK, EMBED), jnp.float32),
            pltpu.VMEM((NBUF, CHUNK, EMBED), jnp.float32),
            pltpu.VMEM((NBUF, CHUNK, EMBED), jnp.float32),
            pltpu.VMEM((NBUF, CHUNK), jnp.float32),
            pltpu.VMEM((NBUF, CHUNK), jnp.float32),
            pltpu.SemaphoreType.DMA,
            pltpu.SemaphoreType.DMA,
            pltpu.SemaphoreType.DMA,
            pltpu.SemaphoreType.DMA,
            pltpu.SemaphoreType.DMA,
            pltpu.SemaphoreType.DMA,
        ],
    )
    p_score, n_score = f(uf, pf, nf, users, items)
    return p_score.reshape(B, L), n_score.reshape(B, L)


# speedup vs baseline: 3.4930x; 1.0017x over previous
import jax
import jax.numpy as jnp
from jax import lax
from jax.experimental import pallas as pl
from jax.experimental.pallas import tpu as pltpu
from jax.experimental.pallas import tpu_sc as plsc

NC = 2
NS = 16
LANES = 16
NW = NC * NS
EMBED = 64
CHUNK = 128
NBUF = 4


def _mf_body(user_hbm, itemp_hbm, itemn_hbm, users_hbm, items_hbm,
             pscore_hbm, nscore_hbm,
             idxu, idxp, idxn, urows, prows, nrows, psc, nsc, *sems):
    gsem = sems[0:NBUF]
    isem = sems[NBUF:2 * NBUF]
    osem = sems[2 * NBUF:3 * NBUF]
    wid = lax.axis_index("s") * NC + lax.axis_index("c")
    n_per_w = user_hbm.shape[0] // NW
    nchunk = n_per_w // CHUNK
    base_w = wid * n_per_w

    def chunk_base(g):
        return pl.multiple_of(base_w + g * CHUNK, CHUNK)

    def fire_idx(g, b, sync):
        base = chunk_base(g)
        if sync:
            pltpu.sync_copy(user_hbm.at[pl.ds(base, CHUNK)], idxu.at[b])
            pltpu.sync_copy(itemp_hbm.at[pl.ds(base, CHUNK)], idxp.at[b])
            pltpu.sync_copy(itemn_hbm.at[pl.ds(base, CHUNK)], idxn.at[b])
        else:
            pltpu.async_copy(user_hbm.at[pl.ds(base, CHUNK)], idxu.at[b], isem[b])
            pltpu.async_copy(itemp_hbm.at[pl.ds(base, CHUNK)], idxp.at[b], isem[b])
            pltpu.async_copy(itemn_hbm.at[pl.ds(base, CHUNK)], idxn.at[b], isem[b])

    def wait_idx(b):
        pltpu.make_async_copy(user_hbm.at[pl.ds(0, CHUNK)], idxu.at[b], isem[b]).wait()
        pltpu.make_async_copy(itemp_hbm.at[pl.ds(0, CHUNK)], idxp.at[b], isem[b]).wait()
        pltpu.make_async_copy(itemn_hbm.at[pl.ds(0, CHUNK)], idxn.at[b], isem[b]).wait()

    def fire_gather(b):
        pltpu.async_copy(users_hbm.at[idxu.at[b]], urows.at[b], gsem[b])
        pltpu.async_copy(items_hbm.at[idxp.at[b]], prows.at[b], gsem[b])
        pltpu.async_copy(items_hbm.at[idxn.at[b]], nrows.at[b], gsem[b])

    def wait_gather(b):
        pltpu.make_async_copy(users_hbm.at[idxu.at[b]], urows.at[b], gsem[b]).wait()
        pltpu.make_async_copy(items_hbm.at[idxp.at[b]], prows.at[b], gsem[b]).wait()
        pltpu.make_async_copy(items_hbm.at[idxn.at[b]], nrows.at[b], gsem[b]).wait()

    def fire_out(g, b):
        base = chunk_base(g)
        pltpu.async_copy(psc.at[b], pscore_hbm.at[pl.ds(base, CHUNK)], osem[b])
        pltpu.async_copy(nsc.at[b], nscore_hbm.at[pl.ds(base, CHUNK)], osem[b])

    def wait_out(b):
        pltpu.make_async_copy(psc.at[b], pscore_hbm.at[pl.ds(0, CHUNK)], osem[b]).wait()
        pltpu.make_async_copy(nsc.at[b], nscore_hbm.at[pl.ds(0, CHUNK)], osem[b]).wait()

    def compute(b):
        last_lane = lax.iota(jnp.int32, LANES) == (LANES - 1)

        def row_body(r, carry):
            accp = None
            accn = None
            for k in range(EMBED // LANES):
                uu = urows[b, r, pl.ds(k * LANES, LANES)]
                pp = prows[b, r, pl.ds(k * LANES, LANES)]
                nn = nrows[b, r, pl.ds(k * LANES, LANES)]
                accp = uu * pp if accp is None else accp + uu * pp
                accn = uu * nn if accn is None else accn + uu * nn
            ridx = jnp.full((LANES,), 0, jnp.int32) + r
            plsc.store_scatter(psc.at[b], [ridx], plsc.cumsum(accp), mask=last_lane)
            plsc.store_scatter(nsc.at[b], [ridx], plsc.cumsum(accn), mask=last_lane)
            return carry

        lax.fori_loop(0, CHUNK, row_body, 0, unroll=4)

    for b in range(NBUF):
        fire_idx(b, b, sync=True)
        fire_gather(b)

    def outer(i, carry):
        for b in range(NBUF):
            g = i * NBUF + b
            wait_gather(b)

            @pl.when(g < nchunk - NBUF)
            def _():
                fire_idx(g + NBUF, b, sync=False)

            @pl.when(g >= NBUF)
            def _():
                wait_out(b)

            compute(b)
            fire_out(g, b)

            @pl.when(g < nchunk - NBUF)
            def _():
                wait_idx(b)
                fire_gather(b)

        return carry

    lax.fori_loop(0, nchunk // NBUF, outer, 0)

    for b in range(NBUF):
        wait_out(b)


def kernel(user, item_p, item_n, users, items):
    B, L = user.shape
    N = B * L
    uf = user.reshape(N)
    pf = item_p.reshape(N)
    nf = item_n.reshape(N)
    mesh = plsc.VectorSubcoreMesh(core_axis_name="c", subcore_axis_name="s")
    f = pl.kernel(
        _mf_body,
        out_type=(jax.ShapeDtypeStruct((N,), jnp.float32),
                  jax.ShapeDtypeStruct((N,), jnp.float32)),
        mesh=mesh,
        compiler_params=pltpu.CompilerParams(needs_layout_passes=False,
                                             use_tc_tiling_on_sc=False),
        scratch_types=[
            pltpu.VMEM((NBUF, CHUNK), jnp.int32),
            pltpu.VMEM((NBUF, CHUNK), jnp.int32),
            pltpu.VMEM((NBUF, CHUNK), jnp.int32),
            pltpu.VMEM((NBUF, CHUNK, EMBED), jnp.float32),
            pltpu.VMEM((NBUF, CHUNK, EMBED), jnp.float32),
            pltpu.VMEM((NBUF, CHUNK, EMBED), jnp.float32),
            pltpu.VMEM((NBUF, CHUNK), jnp.float32),
            pltpu.VMEM((NBUF, CHUNK), jnp.float32),
        ] + [pltpu.SemaphoreType.DMA] * (3 * NBUF),
    )
    p_score, n_score = f(uf, pf, nf, users, items)
    return p_score.reshape(B, L), n_score.reshape(B, L)
